# Initial kernel scaffold; baseline (speedup 1.0000x reference)
import jax
import jax.numpy as jnp
from jax.experimental import pallas as pl


def kernel(user_ids, ID_embeddings):
    raise NotImplementedError("write your pallas kernel here")


# speedup vs baseline: 4.5464x; 4.5464x over previous
import functools

import jax
import jax.numpy as jnp
from jax import lax
from jax.experimental import pallas as pl
from jax.experimental.pallas import tpu as pltpu
from jax.experimental.pallas import tpu_sc as plsc

B = 4096
H = 50
TOTAL = B * H
D = 64
NC = 2
NS = 16
NW = NC * NS
PER_W = TOTAL // NW
CHUNK = 800
NCHUNK = PER_W // CHUNK


@functools.partial(
    pl.kernel,
    mesh=plsc.VectorSubcoreMesh(core_axis_name="c", subcore_axis_name="s"),
    out_type=jax.ShapeDtypeStruct((TOTAL, D), jnp.float32),
    scratch_types=[
        pltpu.VMEM((CHUNK,), jnp.int32),
        pltpu.VMEM((CHUNK, D), jnp.float32),
        pltpu.SemaphoreType.DMA,
    ],
    compiler_params=pltpu.CompilerParams(use_tc_tiling_on_sc=False),
)
def _gather_rows(idx_hbm, table_hbm, out_hbm, idx_v, rows_v, sem):
    wid = lax.axis_index("s") * NC + lax.axis_index("c")
    base = wid * PER_W
    for g in range(NCHUNK):
        off = base + g * CHUNK
        pltpu.sync_copy(idx_hbm.at[pl.ds(off, CHUNK)], idx_v)
        pltpu.async_copy(table_hbm.at[idx_v], rows_v, sem).wait()
        pltpu.sync_copy(rows_v, out_hbm.at[pl.ds(off, CHUNK)])


def kernel(user_ids, ID_embeddings):
    idx = user_ids.reshape(-1).astype(jnp.int32)
    out = _gather_rows(idx, ID_embeddings)
    return out.reshape(B, H, D)


# speedup vs baseline: 4.6509x; 1.0230x over previous
import functools

import jax
import jax.numpy as jnp
from jax import lax
from jax.experimental import pallas as pl
from jax.experimental.pallas import tpu as pltpu
from jax.experimental.pallas import tpu_sc as plsc

B = 4096
H = 50
TOTAL = B * H
D = 64
NC = 2
NS = 16
NW = NC * NS
PER_W = TOTAL // NW
CHUNK = 800
NCHUNK = PER_W // CHUNK


@functools.partial(
    pl.kernel,
    mesh=plsc.VectorSubcoreMesh(core_axis_name="c", subcore_axis_name="s"),
    out_type=jax.ShapeDtypeStruct((TOTAL, D), jnp.float32),
    scratch_types=[
        pltpu.VMEM((CHUNK,), jnp.int32),
        pltpu.VMEM((CHUNK,), jnp.int32),
        pltpu.VMEM((CHUNK, D), jnp.float32),
        pltpu.VMEM((CHUNK, D), jnp.float32),
        pltpu.SemaphoreType.DMA,
        pltpu.SemaphoreType.DMA,
        pltpu.SemaphoreType.DMA,
        pltpu.SemaphoreType.DMA,
        pltpu.SemaphoreType.DMA,
        pltpu.SemaphoreType.DMA,
    ],
    compiler_params=pltpu.CompilerParams(use_tc_tiling_on_sc=False),
)
def _gather_rows(idx_hbm, table_hbm, out_hbm, idx0, idx1, rows0, rows1,
                 isem0, isem1, gsem0, gsem1, ssem0, ssem1):
    wid = lax.axis_index("s") * NC + lax.axis_index("c")
    base = wid * PER_W
    idxs = (idx0, idx1)
    bufs = (rows0, rows1)
    isems = (isem0, isem1)
    gsems = (gsem0, gsem1)
    ssems = (ssem0, ssem1)

    def idxload(g):
        return pltpu.make_async_copy(
            idx_hbm.at[pl.ds(base + g * CHUNK, CHUNK)],
            idxs[g % 2], isems[g % 2])

    def gather(g):
        return pltpu.make_async_copy(
            table_hbm.at[idxs[g % 2]], bufs[g % 2], gsems[g % 2])

    def store(g):
        return pltpu.make_async_copy(
            bufs[g % 2], out_hbm.at[pl.ds(base + g * CHUNK, CHUNK)],
            ssems[g % 2])

    idxload(0).start()
    idxload(1).start()
    idxload(0).wait()
    gather(0).start()
    for g in range(NCHUNK):
        if g + 1 < NCHUNK:
            idxload(g + 1).wait()
            if g >= 1:
                store(g - 1).wait()
            gather(g + 1).start()
        gather(g).wait()
        if g + 2 < NCHUNK:
            idxload(g + 2).start()
        store(g).start()
    store(NCHUNK - 1).wait()


def kernel(user_ids, ID_embeddings):
    idx = user_ids.reshape(-1).astype(jnp.int32)
    out = _gather_rows(idx, ID_embeddings)
    return out.reshape(B, H, D)


# speedup vs baseline: 5.8195x; 1.2513x over previous
import functools

import jax
import jax.numpy as jnp
from jax import lax
from jax.experimental import pallas as pl
from jax.experimental.pallas import tpu as pltpu
from jax.experimental.pallas import tpu_sc as plsc

B = 4096
H = 50
TOTAL = B * H
D = 64
DP = 128
VP = 100008
NC = 2
NS = 16
NW = NC * NS
B_PER_W = B // NW
PER_W = TOTAL // NW
BCH = 8
CHUNK = BCH * H
NCHUNK = PER_W // CHUNK


@functools.partial(
    pl.kernel,
    mesh=plsc.VectorSubcoreMesh(core_axis_name="c", subcore_axis_name="s"),
    out_type=jax.ShapeDtypeStruct((B, H, DP), jnp.float32),
    scratch_types=[
        pltpu.VMEM((CHUNK,), jnp.int32),
        pltpu.VMEM((CHUNK,), jnp.int32),
        pltpu.VMEM((CHUNK, DP), jnp.float32),
        pltpu.VMEM((CHUNK, DP), jnp.float32),
        pltpu.SemaphoreType.DMA,
        pltpu.SemaphoreType.DMA,
        pltpu.SemaphoreType.DMA,
        pltpu.SemaphoreType.DMA,
        pltpu.SemaphoreType.DMA,
        pltpu.SemaphoreType.DMA,
    ],
)
def _gather_rows(idx_hbm, table_hbm, out_hbm, idx0, idx1, rows0, rows1,
                 isem0, isem1, gsem0, gsem1, ssem0, ssem1):
    wid = lax.axis_index("s") * NC + lax.axis_index("c")
    base = wid * PER_W
    bbase = wid * B_PER_W
    idxs = (idx0, idx1)
    bufs = (rows0, rows1)
    isems = (isem0, isem1)
    gsems = (gsem0, gsem1)
    ssems = (ssem0, ssem1)

    def idxload(g):
        return pltpu.make_async_copy(
            idx_hbm.at[pl.ds(base + g * CHUNK, CHUNK)],
            idxs[g % 2], isems[g % 2])

    def gather(g):
        return pltpu.make_async_copy(
            table_hbm.at[idxs[g % 2]], bufs[g % 2], gsems[g % 2])

    def stores(g):
        return [
            pltpu.make_async_copy(
                bufs[g % 2].at[pl.ds(i * H, H)],
                out_hbm.at[bbase + g * BCH + i],
                ssems[g % 2])
            for i in range(BCH)
        ]

    idxload(0).start()
    idxload(1).start()
    idxload(0).wait()
    gather(0).start()
    for g in range(NCHUNK):
        if g + 1 < NCHUNK:
            idxload(g + 1).wait()
            if g >= 1:
                for s in stores(g - 1):
                    s.wait()
            gather(g + 1).start()
        gather(g).wait()
        if g + 2 < NCHUNK:
            idxload(g + 2).start()
        for s in stores(g):
            s.start()
    for s in stores(NCHUNK - 1):
        s.wait()


def kernel(user_ids, ID_embeddings):
    idx = user_ids.reshape(-1).astype(jnp.int32)
    table = jnp.pad(ID_embeddings,
                    ((0, VP - ID_embeddings.shape[0]), (0, DP - D)))
    out = _gather_rows(idx, table)
    return out[:, :, :D]
